# baseline (speedup 1.0000x reference)
import functools
import jax
import jax.numpy as jnp
from jax import lax
from jax.experimental import pallas as pl
from jax.experimental.pallas import tpu as pltpu
from jax.experimental.pallas import tpu_sc as plsc

EMBED = 64
SCALE = 8.0
SPLIT = 128


def _sc_embed(inputs, table):
    nb, nt = inputs.shape
    rem = nt - SPLIT

    mesh = plsc.VectorSubcoreMesh(core_axis_name="c", subcore_axis_name="s")
    info = plsc.get_sparse_core_info()
    nc = info.num_cores
    nw = nc * info.num_subcores
    rows_per_w = nb // nw

    @functools.partial(
        pl.kernel,
        out_type=jax.ShapeDtypeStruct((nb, nt, EMBED), jnp.float32),
        mesh=mesh,
        compiler_params=pltpu.CompilerParams(use_tc_tiling_on_sc=False),
        scratch_types=[
            pltpu.VMEM((rows_per_w, nt), jnp.int32),
            [pltpu.VMEM((SPLIT, EMBED), jnp.float32) for _ in range(2)],
            [pltpu.VMEM((rem, EMBED), jnp.float32) for _ in range(2)],
            [pltpu.SemaphoreType.DMA for _ in range(4)],
            [pltpu.SemaphoreType.DMA for _ in range(4)],
        ],
    )
    def k(idx_hbm, table_hbm, out_hbm, idx_v, bufa, bufb, gsems, wsems):
        wid = lax.axis_index("s") * nc + lax.axis_index("c")
        base = wid * rows_per_w
        pltpu.sync_copy(idx_hbm.at[pl.ds(base, rows_per_w)], idx_v)

        def start_gather(r, s):
            pltpu.async_copy(
                table_hbm.at[idx_v.at[r, pl.ds(0, SPLIT)]], bufa[s],
                gsems[2 * s])
            pltpu.async_copy(
                table_hbm.at[idx_v.at[r, pl.ds(SPLIT, rem)]], bufb[s],
                gsems[2 * s + 1])

        def wait_gather(s):
            pltpu.make_async_copy(
                table_hbm.at[idx_v.at[0, pl.ds(0, SPLIT)]], bufa[s],
                gsems[2 * s]).wait()
            pltpu.make_async_copy(
                table_hbm.at[idx_v.at[0, pl.ds(SPLIT, rem)]], bufb[s],
                gsems[2 * s + 1]).wait()

        def start_write(r, s):
            row = base + r
            pltpu.async_copy(
                bufa[s], out_hbm.at[row, pl.ds(0, SPLIT)], wsems[2 * s])
            pltpu.async_copy(
                bufb[s], out_hbm.at[row, pl.ds(SPLIT, rem)], wsems[2 * s + 1])

        def wait_write(s):
            pltpu.make_async_copy(
                bufa[s], out_hbm.at[0, pl.ds(0, SPLIT)], wsems[2 * s]).wait()
            pltpu.make_async_copy(
                bufb[s], out_hbm.at[0, pl.ds(SPLIT, rem)],
                wsems[2 * s + 1]).wait()

        def scale(s):
            ba, bb = bufa[s], bufb[s]

            @plsc.parallel_loop(0, SPLIT, unroll=4)
            def _(r):
                for j in range(EMBED // 16):
                    sl = pl.ds(16 * j, 16)
                    ba[r, sl] = ba[r, sl] * SCALE

            @plsc.parallel_loop(0, rem, unroll=4)
            def _(r):
                for j in range(EMBED // 16):
                    sl = pl.ds(16 * j, 16)
                    bb[r, sl] = bb[r, sl] * SCALE

        start_gather(0, 0)

        def pair(p, _):
            for s in range(2):
                r = 2 * p + s

                @pl.when(r + 1 < rows_per_w)
                def _():
                    @pl.when(r >= 1)
                    def _():
                        wait_write(1 - s)

                    start_gather(r + 1, 1 - s)

                wait_gather(s)
                scale(s)
                start_write(r, s)
            return _

        lax.fori_loop(0, rows_per_w // 2, pair, None)
        for s in range(2):
            wait_write(s)

    return k(inputs, table)


def kernel(inputs, table):
    return _sc_embed(inputs, table)


# speedup vs baseline: 1.1302x; 1.1302x over previous
import functools
import jax
import jax.numpy as jnp
from jax import lax
from jax.experimental import pallas as pl
from jax.experimental.pallas import tpu as pltpu
from jax.experimental.pallas import tpu_sc as plsc

EMBED = 64
SCALE = 8.0
CHUNK = 128
NBUF = 4


def _sc_embed(idx3, table2):
    nw, n_chunks, _ = idx3.shape
    b_per_w = n_chunks * CHUNK
    B = nw * b_per_w

    mesh = plsc.VectorSubcoreMesh(core_axis_name="c", subcore_axis_name="s")
    info = plsc.get_sparse_core_info()
    nc = info.num_cores

    @functools.partial(
        pl.kernel,
        out_type=jax.ShapeDtypeStruct((B, 2 * EMBED), jnp.float32),
        mesh=mesh,
        compiler_params=pltpu.CompilerParams(use_tc_tiling_on_sc=False),
        scratch_types=[
            pltpu.VMEM((n_chunks, CHUNK), jnp.int32),
            pltpu.VMEM((n_chunks, CHUNK), jnp.int32),
            [pltpu.VMEM((CHUNK, 2 * EMBED), jnp.float32) for _ in range(NBUF)],
            [pltpu.SemaphoreType.DMA for _ in range(NBUF)],
            [pltpu.SemaphoreType.DMA for _ in range(NBUF)],
        ],
    )
    def k(idx_hbm, table_hbm, out_hbm, idx_v, idx2_v, bufs, gsems, wsems):
        wid = lax.axis_index("s") * nc + lax.axis_index("c")
        base = wid * b_per_w
        pltpu.sync_copy(idx_hbm.at[wid], idx_v)

        @plsc.parallel_loop(0, n_chunks, unroll=2)
        def _(c):
            for j in range(CHUNK // 16):
                sl = pl.ds(16 * j, 16)
                idx2_v[c, sl] = lax.shift_right_logical(idx_v[c, sl], 1)

        def start_gather(i, slot):
            pltpu.async_copy(
                table_hbm.at[idx2_v.at[i]], bufs[slot], gsems[slot])

        def wait_gather(slot):
            pltpu.make_async_copy(
                table_hbm.at[idx2_v.at[0]], bufs[slot], gsems[slot]).wait()

        def start_write(i, slot):
            pltpu.async_copy(
                bufs[slot], out_hbm.at[pl.ds(base + i * CHUNK, CHUNK)],
                wsems[slot])

        def wait_write(slot):
            pltpu.make_async_copy(
                bufs[slot], out_hbm.at[pl.ds(base, CHUNK)],
                wsems[slot]).wait()

        def select_scale(i, slot):
            buf = bufs[slot]

            @plsc.parallel_loop(0, CHUNK // 16, unroll=1)
            def _(g):
                offs = (idx_v[i, pl.ds(16 * g, 16)] & 1) * EMBED
                for r16 in range(16):
                    r = 16 * g + r16
                    off = offs[r16]
                    for j in range(EMBED // 16):
                        dst = pl.ds(16 * j, 16)
                        src = pl.ds(off + 16 * j, 16)
                        buf[r, dst] = buf[r, src] * SCALE

        start_gather(0, 0)
        start_gather(1, 1)

        def quad(p, _):
            for b in range(NBUF):
                i = p * NBUF + b
                nslot = (b + 2) % NBUF

                @pl.when(i + 2 < n_chunks)
                def _():
                    @pl.when(i >= 2)
                    def _():
                        wait_write(nslot)

                    start_gather(i + 2, nslot)

                wait_gather(b)
                select_scale(i, b)
                start_write(i, b)
            return _

        lax.fori_loop(0, n_chunks // NBUF, quad, None)
        for b in range(NBUF):
            wait_write(b)

    return k(idx3, table2)


def kernel(inputs, table):
    nb, nt = inputs.shape
    B = nb * nt
    voc, _ = table.shape
    info = plsc.get_sparse_core_info()
    nw = info.num_cores * info.num_subcores
    n_chunks = B // (nw * CHUNK)
    idx3 = inputs.reshape(nw, n_chunks, CHUNK)
    table2 = table.reshape(voc // 2, 2 * EMBED)
    out = _sc_embed(idx3, table2)
    return out.reshape(nb, nt, 2 * EMBED)[:, :, :EMBED]


# speedup vs baseline: 1.2370x; 1.0945x over previous
import functools
import jax
import jax.numpy as jnp
from jax import lax
from jax.experimental import pallas as pl
from jax.experimental.pallas import tpu as pltpu
from jax.experimental.pallas import tpu_sc as plsc

EMBED = 64
SCALE = 8.0
CHUNK = 128
NBUF = 4


def _sc_embed(idx3, table2):
    nw, n_chunks, _ = idx3.shape
    b_per_w = n_chunks * CHUNK
    B = nw * b_per_w

    mesh = plsc.VectorSubcoreMesh(core_axis_name="c", subcore_axis_name="s")
    info = plsc.get_sparse_core_info()
    nc = info.num_cores

    @functools.partial(
        pl.kernel,
        out_type=jax.ShapeDtypeStruct((B, 2 * EMBED), jnp.float32),
        mesh=mesh,
        compiler_params=pltpu.CompilerParams(use_tc_tiling_on_sc=False),
        scratch_types=[
            pltpu.VMEM((n_chunks, CHUNK), jnp.int32),
            [pltpu.VMEM((CHUNK, 2 * EMBED), jnp.float32) for _ in range(NBUF)],
            [pltpu.SemaphoreType.DMA for _ in range(NBUF)],
            [pltpu.SemaphoreType.DMA for _ in range(NBUF)],
        ],
    )
    def k(idx_hbm, table_hbm, out_hbm, idx_v, bufs, gsems, wsems):
        wid = lax.axis_index("s") * nc + lax.axis_index("c")
        base = wid * b_per_w
        pltpu.sync_copy(idx_hbm.at[wid], idx_v)

        def start_gather(i, slot):
            pltpu.async_copy(
                table_hbm.at[idx_v.at[i]], bufs[slot], gsems[slot])

        def wait_gather(slot):
            pltpu.make_async_copy(
                table_hbm.at[idx_v.at[0]], bufs[slot], gsems[slot]).wait()

        def start_write(i, slot):
            pltpu.async_copy(
                bufs[slot], out_hbm.at[pl.ds(base + i * CHUNK, CHUNK)],
                wsems[slot])

        def wait_write(slot):
            pltpu.make_async_copy(
                bufs[slot], out_hbm.at[pl.ds(base, CHUNK)],
                wsems[slot]).wait()

        def select_scale(i, slot):
            buf = bufs[slot]

            @plsc.parallel_loop(0, CHUNK, unroll=4)
            def _(r):
                for j in range(EMBED // 16):
                    sl = pl.ds(16 * j, 16)
                    buf[r, sl] = buf[r, sl] * SCALE

        start_gather(0, 0)
        start_gather(1, 1)

        def quad(p, _):
            for b in range(NBUF):
                i = p * NBUF + b
                nslot = (b + 2) % NBUF

                @pl.when(i + 2 < n_chunks)
                def _():
                    @pl.when(i >= 2)
                    def _():
                        wait_write(nslot)

                    start_gather(i + 2, nslot)

                wait_gather(b)
                select_scale(i, b)
                start_write(i, b)
            return _

        lax.fori_loop(0, n_chunks // NBUF, quad, None)
        for b in range(NBUF):
            wait_write(b)

    return k(idx3, table2)


def kernel(inputs, table):
    nb, nt = inputs.shape
    B = nb * nt
    voc, _ = table.shape
    info = plsc.get_sparse_core_info()
    nw = info.num_cores * info.num_subcores
    n_chunks = B // (nw * CHUNK)
    idx3 = inputs.reshape(nw, n_chunks, CHUNK)
    table2 = jnp.pad(table, ((0, 0), (0, EMBED)))
    out = _sc_embed(idx3, table2)
    return out.reshape(nb, nt, 2 * EMBED)[:, :, :EMBED]


# speedup vs baseline: 1.4427x; 1.1663x over previous
import functools
import jax
import jax.numpy as jnp
from jax import lax
from jax.experimental import pallas as pl
from jax.experimental.pallas import tpu as pltpu
from jax.experimental.pallas import tpu_sc as plsc

EMBED = 64
SCALE = 8.0
CHUNK = 128
NBUF = 4


def _sc_embed(idx3, table2):
    nw, n_chunks, _ = idx3.shape
    b_per_w = n_chunks * CHUNK
    B = nw * b_per_w

    mesh = plsc.VectorSubcoreMesh(core_axis_name="c", subcore_axis_name="s")
    info = plsc.get_sparse_core_info()
    nc = info.num_cores

    @functools.partial(
        pl.kernel,
        out_type=jax.ShapeDtypeStruct((B, 2 * EMBED), jnp.float32),
        mesh=mesh,
        compiler_params=pltpu.CompilerParams(use_tc_tiling_on_sc=False),
        scratch_types=[
            pltpu.VMEM((n_chunks, CHUNK), jnp.int32),
            pltpu.VMEM((n_chunks, CHUNK), jnp.int32),
            [pltpu.VMEM((CHUNK, EMBED), jnp.float32) for _ in range(NBUF)],
            [pltpu.SemaphoreType.DMA for _ in range(NBUF)],
            [pltpu.SemaphoreType.DMA for _ in range(NBUF)],
        ],
    )
    def k(idx_hbm, table_hbm, out_hbm, idx_v, idx2_v, bufs, gsems, wsems):
        wid = lax.axis_index("s") * nc + lax.axis_index("c")
        base = wid * b_per_w
        pltpu.sync_copy(idx_hbm.at[wid], idx_v)

        @plsc.parallel_loop(0, n_chunks, unroll=2)
        def _(c):
            for j in range(CHUNK // 16):
                sl = pl.ds(16 * j, 16)
                idx2_v[c, sl] = lax.shift_left(idx_v[c, sl], 1)

        def start_gather(i, slot):
            pltpu.async_copy(
                table_hbm.at[idx2_v.at[i]], bufs[slot], gsems[slot])

        def wait_gather(slot):
            pltpu.make_async_copy(
                table_hbm.at[idx2_v.at[0]], bufs[slot], gsems[slot]).wait()

        def start_write(i, slot):
            pltpu.async_copy(
                bufs[slot],
                out_hbm.at[pl.ds(base + i * CHUNK, CHUNK), pl.ds(0, EMBED)],
                wsems[slot])

        def wait_write(slot):
            pltpu.make_async_copy(
                bufs[slot],
                out_hbm.at[pl.ds(base, CHUNK), pl.ds(0, EMBED)],
                wsems[slot]).wait()

        def select_scale(i, slot):
            buf = bufs[slot]

            @plsc.parallel_loop(0, CHUNK, unroll=4)
            def _(r):
                for j in range(EMBED // 16):
                    sl = pl.ds(16 * j, 16)
                    buf[r, sl] = buf[r, sl] * SCALE

        start_gather(0, 0)
        start_gather(1, 1)

        def quad(p, _):
            for b in range(NBUF):
                i = p * NBUF + b
                nslot = (b + 2) % NBUF

                @pl.when(i + 2 < n_chunks)
                def _():
                    @pl.when(i >= 2)
                    def _():
                        wait_write(nslot)

                    start_gather(i + 2, nslot)

                wait_gather(b)
                select_scale(i, b)
                start_write(i, b)
            return _

        lax.fori_loop(0, n_chunks // NBUF, quad, None)
        for b in range(NBUF):
            wait_write(b)

    return k(idx3, table2)


def kernel(inputs, table):
    nb, nt = inputs.shape
    B = nb * nt
    voc, _ = table.shape
    info = plsc.get_sparse_core_info()
    nw = info.num_cores * info.num_subcores
    n_chunks = B // (nw * CHUNK)
    idx3 = inputs.reshape(nw, n_chunks, CHUNK)
    table2 = jnp.pad(table, ((0, 0), (0, EMBED))).reshape(2 * voc, EMBED)
    out = _sc_embed(idx3, table2)
    return out.reshape(nb, nt, 2 * EMBED)[:, :, :EMBED]


# speedup vs baseline: 1.4440x; 1.0010x over previous
import functools
import jax
import jax.numpy as jnp
from jax import lax
from jax.experimental import pallas as pl
from jax.experimental.pallas import tpu as pltpu
from jax.experimental.pallas import tpu_sc as plsc

EMBED = 64
SCALE = 8.0
CHUNK = 128
NBUF = 5
AHEAD = 3


def _sc_embed(idx3, table2):
    nw, n_chunks, _ = idx3.shape
    b_per_w = n_chunks * CHUNK
    B = nw * b_per_w

    mesh = plsc.VectorSubcoreMesh(core_axis_name="c", subcore_axis_name="s")
    info = plsc.get_sparse_core_info()
    nc = info.num_cores

    @functools.partial(
        pl.kernel,
        out_type=jax.ShapeDtypeStruct((B, 2 * EMBED), jnp.float32),
        mesh=mesh,
        compiler_params=pltpu.CompilerParams(use_tc_tiling_on_sc=False),
        scratch_types=[
            pltpu.VMEM((n_chunks, CHUNK), jnp.int32),
            pltpu.VMEM((n_chunks, CHUNK), jnp.int32),
            [pltpu.VMEM((CHUNK, EMBED), jnp.float32) for _ in range(NBUF)],
            [pltpu.SemaphoreType.DMA for _ in range(NBUF)],
            [pltpu.SemaphoreType.DMA for _ in range(NBUF)],
        ],
    )
    def k(idx_hbm, table_hbm, out_hbm, idx_v, idx2_v, bufs, gsems, wsems):
        wid = lax.axis_index("s") * nc + lax.axis_index("c")
        base = wid * b_per_w
        pltpu.sync_copy(idx_hbm.at[wid], idx_v)

        @plsc.parallel_loop(0, n_chunks, unroll=2)
        def _(c):
            for j in range(CHUNK // 16):
                sl = pl.ds(16 * j, 16)
                idx2_v[c, sl] = lax.shift_left(idx_v[c, sl], 1)

        def start_gather(i, slot):
            pltpu.async_copy(
                table_hbm.at[idx2_v.at[i]], bufs[slot], gsems[slot])

        def wait_gather(slot):
            pltpu.make_async_copy(
                table_hbm.at[idx2_v.at[0]], bufs[slot], gsems[slot]).wait()

        def start_write(i, slot):
            pltpu.async_copy(
                bufs[slot],
                out_hbm.at[pl.ds(base + i * CHUNK, CHUNK), pl.ds(0, EMBED)],
                wsems[slot])

        def wait_write(slot):
            pltpu.make_async_copy(
                bufs[slot],
                out_hbm.at[pl.ds(base, CHUNK), pl.ds(0, EMBED)],
                wsems[slot]).wait()

        def select_scale(i, slot):
            buf = bufs[slot]

            @plsc.parallel_loop(0, CHUNK, unroll=4)
            def _(r):
                for j in range(EMBED // 16):
                    sl = pl.ds(16 * j, 16)
                    buf[r, sl] = buf[r, sl] * SCALE

        for j in range(AHEAD):
            start_gather(j, j)

        def group(p, _):
            for b in range(NBUF):
                i = p * NBUF + b
                nslot = (b + AHEAD) % NBUF

                @pl.when(i + AHEAD < n_chunks)
                def _():
                    @pl.when(i >= NBUF - AHEAD)
                    def _():
                        wait_write(nslot)

                    start_gather(i + AHEAD, nslot)

                wait_gather(b)
                select_scale(i, b)
                start_write(i, b)
            return _

        lax.fori_loop(0, n_chunks // NBUF, group, None)
        for b in range(NBUF):
            wait_write(b)

    return k(idx3, table2)


def kernel(inputs, table):
    nb, nt = inputs.shape
    B = nb * nt
    voc, _ = table.shape
    info = plsc.get_sparse_core_info()
    nw = info.num_cores * info.num_subcores
    n_chunks = B // (nw * CHUNK)
    idx3 = inputs.reshape(nw, n_chunks, CHUNK)
    table2 = jnp.pad(table, ((0, 0), (0, EMBED))).reshape(2 * voc, EMBED)
    out = _sc_embed(idx3, table2)
    return out.reshape(nb, nt, 2 * EMBED)[:, :, :EMBED]
